# baseline (speedup 1.0000x reference)
import jax
import jax.numpy as jnp
from jax.experimental import pallas as pl
from jax.experimental.pallas import tpu as pltpu

B, N, K = 128, 32768, 10
SUB, LANE = 256, 128
GRP = 8 * LANE
NGRP = N * K // GRP
NB = 8
BG = NGRP // NB


def _body(x_ref, out_ref, t_smem, s_ref):
    nb = pl.program_id(1)

    @pl.when(nb == 0)
    def _topk():
        s_ref[...] = x_ref[0]
        flat = (jax.lax.broadcasted_iota(jnp.int32, (SUB, LANE), 0) * LANE
                + jax.lax.broadcasted_iota(jnp.int32, (SUB, LANE), 1))
        idxs = []
        for _ in range(K):
            s = s_ref[...]
            v = jnp.max(s)
            i = jnp.min(jnp.where(s == v, flat, N))
            idxs.append(i)
            s_ref[...] = jnp.where(flat == i, -jnp.inf, s)
        for k in range(K):
            rank = jnp.int32(0)
            for m in range(K):
                if m != k:
                    rank = rank + (idxs[m] < idxs[k]).astype(jnp.int32)
            t_smem[k] = idxs[k] * K + rank

    out_ref[...] = jnp.zeros_like(out_ref)
    p0 = nb * (BG * GRP)
    sub = jax.lax.broadcasted_iota(jnp.int32, (1, 8, LANE), 1)
    lane = jax.lax.broadcasted_iota(jnp.int32, (1, 8, LANE), 2)
    for j in range(K):
        local = t_smem[j] - p0

        @pl.when((local >= 0) & (local < BG * GRP))
        def _():
            rg = local // GRP
            q = local - rg * GRP
            qr = q // LANE
            qc = q - qr * LANE
            out_ref[0, pl.ds(rg, 1)] = out_ref[0, pl.ds(rg, 1)] + (
                (sub == qr) & (lane == qc)).astype(jnp.float32)


def kernel(x):
    x3 = x.reshape(B, SUB, LANE)
    out = pl.pallas_call(
        _body,
        grid=(B, NB),
        in_specs=[pl.BlockSpec((1, SUB, LANE), lambda b, nb: (b, 0, 0))],
        out_specs=pl.BlockSpec((1, BG, 8, LANE), lambda b, nb: (b, nb, 0, 0)),
        out_shape=jax.ShapeDtypeStruct((B, NGRP, 8, LANE), jnp.float32),
        scratch_shapes=[
            pltpu.SMEM((K,), jnp.int32),
            pltpu.VMEM((SUB, LANE), jnp.float32),
        ],
    )(x3)
    return out.reshape(B, N, K)


# speedup vs baseline: 1.3753x; 1.3753x over previous
import jax
import jax.numpy as jnp
from jax.experimental import pallas as pl
from jax.experimental.pallas import tpu as pltpu

B, N, K = 128, 32768, 10
LANE = 128
GRP = 8 * LANE
NGRP = N * K // GRP
RPB = 8
NEG = float('-inf')


def _topk_body(x_ref, tgt_ref, zout_ref, zs_ref, s_ref, sem):
    step = pl.program_id(0)

    @pl.when(step == 0)
    def _():
        zs_ref[...] = jnp.zeros_like(zs_ref)

    copies = [
        pltpu.make_async_copy(zs_ref, zout_ref.at[step * RPB + r], sem)
        for r in range(RPB)
    ]
    for c in copies:
        c.start()

    li = jax.lax.broadcasted_iota(jnp.int32, (RPB, N), 1)
    s_ref[...] = x_ref[...]
    idxs = []
    for _ in range(K):
        s = s_ref[...]
        v = jnp.max(s, axis=1, keepdims=True)
        i = jnp.min(jnp.where(s == v, li, N), axis=1, keepdims=True)
        idxs.append(i)
        s_ref[...] = jnp.where(li == i, NEG, s)
    cols = []
    for k in range(K):
        rank = jnp.zeros((RPB, 1), jnp.int32)
        for m in range(K):
            if m != k:
                rank = rank + (idxs[m] < idxs[k]).astype(jnp.int32)
        cols.append(idxs[k] * K + rank)
    tgt_ref[...] = jnp.concatenate(cols + cols[-1:] * (16 - K), axis=1)

    for c in copies:
        c.wait()


def _patch_body(tref, x_any, out_ref):
    b = pl.program_id(0)
    j = pl.program_id(1)
    g = tref[b, j] // GRP
    sub = jax.lax.broadcasted_iota(jnp.int32, (1, 1, 8, LANE), 2)
    lane = jax.lax.broadcasted_iota(jnp.int32, (1, 1, 8, LANE), 3)
    acc = jnp.zeros((1, 1, 8, LANE), jnp.float32)
    for m in range(K):
        t = tref[b, m]
        tg = t // GRP
        q = t - tg * GRP
        qr = q // LANE
        qc = q - qr * LANE
        acc = acc + ((tg == g) & (sub == qr) & (lane == qc)).astype(jnp.float32)
    out_ref[...] = acc


def kernel(x):
    tgt, zbuf = pl.pallas_call(
        _topk_body,
        grid=(B // RPB,),
        in_specs=[pl.BlockSpec((RPB, N), lambda i: (i, 0))],
        out_specs=[
            pl.BlockSpec((RPB, 16), lambda i: (i, 0)),
            pl.BlockSpec(memory_space=pl.ANY),
        ],
        out_shape=[
            jax.ShapeDtypeStruct((B, 16), jnp.int32),
            jax.ShapeDtypeStruct((B, NGRP, 8, LANE), jnp.float32),
        ],
        scratch_shapes=[
            pltpu.VMEM((NGRP, 8, LANE), jnp.float32),
            pltpu.VMEM((RPB, N), jnp.float32),
            pltpu.SemaphoreType.DMA,
        ],
    )(x)

    out = pl.pallas_call(
        _patch_body,
        grid_spec=pltpu.PrefetchScalarGridSpec(
            num_scalar_prefetch=1,
            grid=(B, K),
            in_specs=[pl.BlockSpec(memory_space=pl.ANY)],
            out_specs=pl.BlockSpec(
                (1, 1, 8, LANE),
                lambda b, j, tref: (b, tref[b, j] // GRP, 0, 0),
            ),
        ),
        out_shape=jax.ShapeDtypeStruct((B, NGRP, 8, LANE), jnp.float32),
        input_output_aliases={1: 0},
    )(tgt, zbuf)
    return out.reshape(B, N, K)


# speedup vs baseline: 7.2169x; 5.2475x over previous
import jax
import jax.numpy as jnp
from jax.experimental import pallas as pl
from jax.experimental.pallas import tpu as pltpu

B, N, K = 128, 32768, 10
RPB = 8
STEPS = B // RPB
W = 512
NEG = float('-inf')


def _body(x_ref, zout_ref, zs_ref, s_ref, seg_ref, t_smem, zsem, psem):
    step = pl.program_id(0)

    @pl.when(step == 0)
    def _():
        zs_ref[...] = jnp.zeros_like(zs_ref)

    zcopies = [
        pltpu.make_async_copy(
            zs_ref, zout_ref.at[pl.ds(j * B + step * RPB, RPB), :], zsem)
        for j in range(K)
    ]
    for c in zcopies:
        c.start()

    li = jax.lax.broadcasted_iota(jnp.int32, (RPB, N), 1)
    s_ref[...] = x_ref[...]
    idxs = []
    for _ in range(K):
        s = s_ref[...]
        v = jnp.max(s, axis=1, keepdims=True)
        i = jnp.min(jnp.where(s == v, li, N), axis=1, keepdims=True)
        idxs.append(i)
        s_ref[...] = jnp.where(li == i, NEG, s)

    ci = jax.lax.broadcasted_iota(jnp.int32, (RPB, W), 1)
    for k in range(K):
        seg_ref[k * RPB:(k + 1) * RPB, :] = (
            ci == idxs[k] % W).astype(jnp.float32)

    ri = jax.lax.broadcasted_iota(jnp.int32, (RPB, 1), 0)
    for k in range(K):
        for r in range(RPB):
            t_smem[k * RPB + r] = jnp.sum(jnp.where(ri == r, idxs[k], 0))

    for c in zcopies:
        c.wait()

    pcopies = []
    for r in range(RPB):
        iscal = [t_smem[k * RPB + r] for k in range(K)]
        for k in range(K):
            rank = (iscal[0] < iscal[k]).astype(jnp.int32) if k else 0
            for m in range(1, K):
                if m != k:
                    rank = rank + (iscal[m] < iscal[k]).astype(jnp.int32)
            c = pltpu.make_async_copy(
                seg_ref.at[k * RPB + r],
                zout_ref.at[rank * B + step * RPB + r,
                            pl.ds((iscal[k] // W) * W, W)],
                psem)
            c.start()
            pcopies.append(c)
    for c in pcopies:
        c.wait()


def kernel(x):
    planes = pl.pallas_call(
        _body,
        grid=(STEPS,),
        in_specs=[pl.BlockSpec((RPB, N), lambda i: (i, 0))],
        out_specs=pl.BlockSpec(memory_space=pl.ANY),
        out_shape=jax.ShapeDtypeStruct((K * B, N), jnp.float32),
        scratch_shapes=[
            pltpu.VMEM((RPB, N), jnp.float32),
            pltpu.VMEM((RPB, N), jnp.float32),
            pltpu.VMEM((K * RPB, W), jnp.float32),
            pltpu.SMEM((K * RPB,), jnp.int32),
            pltpu.SemaphoreType.DMA,
            pltpu.SemaphoreType.DMA,
        ],
    )(x)
    return jnp.transpose(planes.reshape(K, B, N), (1, 2, 0))


# speedup vs baseline: 14.7813x; 2.0482x over previous
import jax
import jax.numpy as jnp
from jax.experimental import pallas as pl
from jax.experimental.pallas import tpu as pltpu

B, N, K = 128, 32768, 10
RPB = 16
STEPS = B // RPB
NSEG = K * RPB
W = 512
NEG = float('-inf')


def _body(x_ref, zout_ref, zs_ref, s_ref, seg_ref, t_smem, zsem, psem):
    step = pl.program_id(0)
    buf = jax.lax.rem(step, 2)

    @pl.when(step == 0)
    def _():
        zs_ref[...] = jnp.zeros_like(zs_ref)

    zcopies = [
        pltpu.make_async_copy(
            zs_ref, zout_ref.at[pl.ds(j * B + step * RPB, RPB), :], zsem)
        for j in range(K)
    ]
    for c in zcopies:
        c.start()

    @pl.when(step > 0)
    def _():
        for _ in range(NSEG):
            pltpu.make_async_copy(
                seg_ref.at[0, 0], zout_ref.at[0, pl.ds(0, W)], psem).wait()

    li = jax.lax.broadcasted_iota(jnp.int32, (RPB, N), 1)
    s_ref[...] = x_ref[...]
    idxs = []
    for _ in range(K):
        s = s_ref[...]
        v = jnp.max(s, axis=1, keepdims=True)
        i = jnp.min(jnp.where(s == v, li, N), axis=1, keepdims=True)
        idxs.append(i)
        s_ref[...] = jnp.where(li == i, NEG, s)

    ci = jax.lax.broadcasted_iota(jnp.int32, (RPB, W), 1)
    for k in range(K):
        seg_ref[buf, k * RPB:(k + 1) * RPB, :] = (
            ci == idxs[k] % W).astype(jnp.float32)

    ri = jax.lax.broadcasted_iota(jnp.int32, (RPB, 1), 0)
    for k in range(K):
        for r in range(RPB):
            t_smem[k * RPB + r] = jnp.sum(jnp.where(ri == r, idxs[k], 0))

    for c in zcopies:
        c.wait()

    pcopies = []
    for r in range(RPB):
        iscal = [t_smem[k * RPB + r] for k in range(K)]
        for k in range(K):
            rank = (iscal[0] < iscal[k]).astype(jnp.int32) if k else 0
            for m in range(1, K):
                if m != k:
                    rank = rank + (iscal[m] < iscal[k]).astype(jnp.int32)
            c = pltpu.make_async_copy(
                seg_ref.at[buf, k * RPB + r],
                zout_ref.at[rank * B + step * RPB + r,
                            pl.ds((iscal[k] // W) * W, W)],
                psem)
            c.start()
            pcopies.append(c)

    @pl.when(step == STEPS - 1)
    def _():
        for c in pcopies:
            c.wait()


def kernel(x):
    planes = pl.pallas_call(
        _body,
        grid=(STEPS,),
        in_specs=[pl.BlockSpec((RPB, N), lambda i: (i, 0))],
        out_specs=pl.BlockSpec(memory_space=pl.ANY),
        out_shape=jax.ShapeDtypeStruct((K * B, N), jnp.float32),
        scratch_shapes=[
            pltpu.VMEM((RPB, N), jnp.float32),
            pltpu.VMEM((RPB, N), jnp.float32),
            pltpu.VMEM((2, NSEG, W), jnp.float32),
            pltpu.SMEM((NSEG,), jnp.int32),
            pltpu.SemaphoreType.DMA,
            pltpu.SemaphoreType.DMA,
        ],
    )(x)
    return jnp.transpose(planes.reshape(K, B, N), (1, 2, 0))


# speedup vs baseline: 15.1128x; 1.0224x over previous
import jax
import jax.numpy as jnp
from jax.experimental import pallas as pl
from jax.experimental.pallas import tpu as pltpu

B, N, K = 128, 32768, 10
RPB = 16
STEPS = B // RPB
NSEG = K * RPB
W = 512
NEG = float('-inf')


def _issue_patches(seg_ref, zout_ref, t_smem, psem, step, buf):
    pcopies = []
    for r in range(RPB):
        iscal = [t_smem[buf, k * RPB + r] for k in range(K)]
        for k in range(K):
            rank = (iscal[0] < iscal[k]).astype(jnp.int32) if k else 0
            for m in range(1, K):
                if m != k:
                    rank = rank + (iscal[m] < iscal[k]).astype(jnp.int32)
            c = pltpu.make_async_copy(
                seg_ref.at[buf, k * RPB + r],
                zout_ref.at[rank * B + step * RPB + r,
                            pl.ds((iscal[k] // W) * W, W)],
                psem.at[buf])
            c.start()
            pcopies.append(c)
    return pcopies


def _body(x_ref, zout_ref, zs_ref, s_ref, seg_ref, t_smem, zsem, psem):
    step = pl.program_id(0)
    buf = jax.lax.rem(step, 2)

    @pl.when(step == 0)
    def _():
        zs_ref[...] = jnp.zeros_like(zs_ref)

    zcopies = [
        pltpu.make_async_copy(
            zs_ref, zout_ref.at[pl.ds(j * B + step * RPB, RPB), :], zsem)
        for j in range(K)
    ]
    for c in zcopies:
        c.start()

    @pl.when(step > 0)
    def _():
        _issue_patches(seg_ref, zout_ref, t_smem, psem, step - 1, 1 - buf)

    @pl.when(step > 1)
    def _():
        for _ in range(NSEG):
            pltpu.make_async_copy(
                seg_ref.at[0, 0], zout_ref.at[0, pl.ds(0, W)],
                psem.at[buf]).wait()

    li = jax.lax.broadcasted_iota(jnp.int32, (RPB, N), 1)
    s_ref[...] = x_ref[...]
    idxs = []
    for _ in range(K):
        s = s_ref[...]
        v = jnp.max(s, axis=1, keepdims=True)
        i = jnp.min(jnp.where(s == v, li, N), axis=1, keepdims=True)
        idxs.append(i)
        s_ref[...] = jnp.where(li == i, NEG, s)

    ci = jax.lax.broadcasted_iota(jnp.int32, (RPB, W), 1)
    for k in range(K):
        seg_ref[buf, k * RPB:(k + 1) * RPB, :] = (
            ci == idxs[k] % W).astype(jnp.float32)

    ri = jax.lax.broadcasted_iota(jnp.int32, (RPB, 1), 0)
    for k in range(K):
        for r in range(RPB):
            t_smem[buf, k * RPB + r] = jnp.sum(jnp.where(ri == r, idxs[k], 0))

    for c in zcopies:
        c.wait()

    @pl.when(step == STEPS - 1)
    def _():
        pc = _issue_patches(seg_ref, zout_ref, t_smem, psem, step, buf)
        for c in pc:
            c.wait()
        for _ in range(NSEG):
            pltpu.make_async_copy(
                seg_ref.at[0, 0], zout_ref.at[0, pl.ds(0, W)],
                psem.at[1 - buf]).wait()


def kernel(x):
    planes = pl.pallas_call(
        _body,
        grid=(STEPS,),
        in_specs=[pl.BlockSpec((RPB, N), lambda i: (i, 0))],
        out_specs=pl.BlockSpec(memory_space=pl.ANY),
        out_shape=jax.ShapeDtypeStruct((K * B, N), jnp.float32),
        scratch_shapes=[
            pltpu.VMEM((RPB, N), jnp.float32),
            pltpu.VMEM((RPB, N), jnp.float32),
            pltpu.VMEM((2, NSEG, W), jnp.float32),
            pltpu.SMEM((2, NSEG), jnp.int32),
            pltpu.SemaphoreType.DMA,
            pltpu.SemaphoreType.DMA((2,)),
        ],
    )(x)
    return jnp.transpose(planes.reshape(K, B, N), (1, 2, 0))


# speedup vs baseline: 16.5192x; 1.0931x over previous
import jax
import jax.numpy as jnp
from jax.experimental import pallas as pl
from jax.experimental.pallas import tpu as pltpu

B, N, K = 128, 32768, 10
RPB = 32
STEPS = B // RPB
NSEG = K * RPB
W = 512
NEG = float('-inf')


def _issue_patches(seg_ref, zout_ref, t_smem, psem, step, buf):
    pcopies = []
    for r in range(RPB):
        iscal = [t_smem[buf, k * RPB + r] for k in range(K)]
        for k in range(K):
            rank = (iscal[0] < iscal[k]).astype(jnp.int32) if k else 0
            for m in range(1, K):
                if m != k:
                    rank = rank + (iscal[m] < iscal[k]).astype(jnp.int32)
            c = pltpu.make_async_copy(
                seg_ref.at[buf, k * RPB + r],
                zout_ref.at[rank * B + step * RPB + r,
                            pl.ds((iscal[k] // W) * W, W)],
                psem.at[buf])
            c.start()
            pcopies.append(c)
    return pcopies


def _body(x_ref, zout_ref, zs_ref, s_ref, seg_ref, t_smem, zsem, psem):
    step = pl.program_id(0)
    buf = jax.lax.rem(step, 2)

    @pl.when(step == 0)
    def _():
        zs_ref[...] = jnp.zeros_like(zs_ref)

    zcopies = [
        pltpu.make_async_copy(
            zs_ref, zout_ref.at[pl.ds(j * B + step * RPB, RPB), :], zsem)
        for j in range(K)
    ]
    for c in zcopies:
        c.start()

    @pl.when(step > 0)
    def _():
        _issue_patches(seg_ref, zout_ref, t_smem, psem, step - 1, 1 - buf)

    @pl.when(step > 1)
    def _():
        for _ in range(NSEG):
            pltpu.make_async_copy(
                seg_ref.at[0, 0], zout_ref.at[0, pl.ds(0, W)],
                psem.at[buf]).wait()

    li = jax.lax.broadcasted_iota(jnp.int32, (RPB, N), 1)
    s_ref[...] = x_ref[...]
    idxs = []
    for _ in range(K):
        s = s_ref[...]
        v = jnp.max(s, axis=1, keepdims=True)
        i = jnp.min(jnp.where(s == v, li, N), axis=1, keepdims=True)
        idxs.append(i)
        s_ref[...] = jnp.where(li == i, NEG, s)

    ci = jax.lax.broadcasted_iota(jnp.int32, (RPB, W), 1)
    for k in range(K):
        seg_ref[buf, k * RPB:(k + 1) * RPB, :] = (
            ci == idxs[k] % W).astype(jnp.float32)

    ri = jax.lax.broadcasted_iota(jnp.int32, (RPB, 1), 0)
    for k in range(K):
        for r in range(RPB):
            t_smem[buf, k * RPB + r] = jnp.sum(jnp.where(ri == r, idxs[k], 0))

    for c in zcopies:
        c.wait()

    @pl.when(step == STEPS - 1)
    def _():
        pc = _issue_patches(seg_ref, zout_ref, t_smem, psem, step, buf)
        for c in pc:
            c.wait()
        for _ in range(NSEG):
            pltpu.make_async_copy(
                seg_ref.at[0, 0], zout_ref.at[0, pl.ds(0, W)],
                psem.at[1 - buf]).wait()


def kernel(x):
    planes = pl.pallas_call(
        _body,
        grid=(STEPS,),
        in_specs=[pl.BlockSpec((RPB, N), lambda i: (i, 0))],
        out_specs=pl.BlockSpec(memory_space=pl.ANY),
        out_shape=jax.ShapeDtypeStruct((K * B, N), jnp.float32),
        scratch_shapes=[
            pltpu.VMEM((RPB, N), jnp.float32),
            pltpu.VMEM((RPB, N), jnp.float32),
            pltpu.VMEM((2, NSEG, W), jnp.float32),
            pltpu.SMEM((2, NSEG), jnp.int32),
            pltpu.SemaphoreType.DMA,
            pltpu.SemaphoreType.DMA((2,)),
        ],
    )(x)
    return jnp.transpose(planes.reshape(K, B, N), (1, 2, 0))
